# baseline (speedup 1.0000x reference)
import functools

import jax
import jax.numpy as jnp
from jax import lax
from jax.experimental import pallas as pl
from jax.experimental.pallas import tpu as pltpu
from jax.experimental.pallas import tpu_sc as plsc

N = 10000
E = 320000
NUM_GRAPHS = 64
FC = 128
BLK = 128
NCORES = 2
NSUB = 16
NBLK = 2560
E_PAD = NBLK * BLK
BLK_PER_SUB = NBLK // (NCORES * NSUB)
ACC_ROWS = 10112
ZR = ACC_ROWS // NSUB
RBLK = 2000
IDXG = 16
ZB = 64


def _vector_mesh():
    return plsc.VectorSubcoreMesh(core_axis_name="c", subcore_axis_name="s")


def _zero_rows(zbuf, acc, z0):
    nfull = ZR // ZB
    for k in range(nfull):
        pltpu.sync_copy(zbuf, acc.at[pl.ds(z0 + k * ZB, ZB)])
    rem = ZR - nfull * ZB
    if rem:
        pltpu.sync_copy(zbuf.at[pl.ds(0, rem)],
                        acc.at[pl.ds(z0 + nfull * ZB, rem)])


def _sc_degree_call(dst_p, ones128, zeros_fc):
    @functools.partial(
        pl.kernel,
        out_type=jax.ShapeDtypeStruct((NCORES * ACC_ROWS, FC), jnp.float32),
        mesh=_vector_mesh(),
        scratch_types=[
            pltpu.VMEM((BLK_PER_SUB, BLK), jnp.int32),
            pltpu.VMEM((BLK, FC), jnp.float32),
            pltpu.VMEM((ZB, FC), jnp.float32),
            pltpu.VMEM_SHARED((ACC_ROWS, FC), jnp.float32),
            pltpu.SemaphoreType.DMA,
        ],
    )
    def deg_kernel(dst_hbm, ones_hbm, zeros_hbm, out_hbm, dst_v, ones_v,
                   zbuf, acc, ssem):
        c = lax.axis_index("c")
        s = lax.axis_index("s")
        b0 = (c * NSUB + s) * BLK_PER_SUB
        pltpu.sync_copy(dst_hbm.at[pl.ds(b0, BLK_PER_SUB)], dst_v)
        pltpu.sync_copy(ones_hbm, ones_v)
        pltpu.sync_copy(zeros_hbm, zbuf)
        z0 = s * ZR
        _zero_rows(zbuf, acc, z0)
        plsc.subcore_barrier()

        @pl.loop(0, BLK_PER_SUB, step=8)
        def _(j):
            for k in range(8):
                pltpu.async_copy(ones_v, acc.at[dst_v.at[j + k]], ssem,
                                 add=True)
            for k in range(8):
                pltpu.make_async_copy(ones_hbm, ones_v, ssem).wait()

        plsc.subcore_barrier()
        pltpu.sync_copy(acc.at[pl.ds(z0, ZR)],
                        out_hbm.at[pl.ds(c * ACC_ROWS + z0, ZR)])

    return deg_kernel(dst_p, ones128, zeros_fc).reshape(NCORES, ACC_ROWS, FC)


def _sc_agg_call(n_chunks, y, src_p, dst_p, zeros_fc):
    @functools.partial(
        pl.kernel,
        out_type=jax.ShapeDtypeStruct(
            (NCORES * n_chunks * ACC_ROWS, FC), jnp.float32),
        mesh=_vector_mesh(),
        scratch_types=[
            pltpu.VMEM((IDXG, BLK), jnp.int32),
            pltpu.VMEM((IDXG, BLK), jnp.int32),
            pltpu.VMEM((BLK, FC), jnp.float32),
            pltpu.VMEM((BLK, FC), jnp.float32),
            pltpu.VMEM((ZB, FC), jnp.float32),
            pltpu.VMEM_SHARED((ACC_ROWS, FC), jnp.float32),
            pltpu.SemaphoreType.DMA,
            pltpu.SemaphoreType.DMA,
            pltpu.SemaphoreType.DMA,
            pltpu.SemaphoreType.DMA,
            pltpu.SemaphoreType.DMA,
        ],
    )
    def agg_kernel(y_hbm, src_hbm, dst_hbm, zeros_hbm, out_hbm,
                   src_v, dst_v, rows0, rows1, zbuf, acc,
                   gsem0, gsem1, ssem0, ssem1, isem):
        c = lax.axis_index("c")
        s = lax.axis_index("s")
        n_grp = BLK_PER_SUB // IDXG
        base = (c * NSUB + s) * BLK_PER_SUB
        pltpu.sync_copy(zeros_hbm, zbuf)

        def drain_scatters():
            pltpu.make_async_copy(y_hbm.at[0].at[pl.ds(0, BLK)], rows0,
                                  ssem0).wait()
            pltpu.make_async_copy(y_hbm.at[0].at[pl.ds(0, BLK)], rows1,
                                  ssem1).wait()

        for ch in range(n_chunks):
            z0 = s * ZR
            _zero_rows(zbuf, acc, z0)
            plsc.subcore_barrier()

            @pl.loop(0, n_grp)
            def _(g):
                @pl.when(g > 0)
                def _():
                    drain_scatters()

                gb = base + g * IDXG
                ld0 = pltpu.async_copy(src_hbm.at[pl.ds(gb, IDXG)], src_v,
                                       isem)
                ld1 = pltpu.async_copy(dst_hbm.at[pl.ds(gb, IDXG)], dst_v,
                                       isem)
                ld0.wait()
                ld1.wait()
                for j in range(0, IDXG, 2):
                    if j > 0:
                        drain_scatters()
                    cp0 = pltpu.async_copy(
                        y_hbm.at[ch].at[src_v.at[j]], rows0, gsem0)
                    cp1 = pltpu.async_copy(
                        y_hbm.at[ch].at[src_v.at[j + 1]], rows1, gsem1)
                    cp0.wait()
                    pltpu.async_copy(rows0, acc.at[dst_v.at[j]], ssem0,
                                     add=True)
                    cp1.wait()
                    pltpu.async_copy(rows1, acc.at[dst_v.at[j + 1]], ssem1,
                                     add=True)

            drain_scatters()
            plsc.subcore_barrier()
            pltpu.sync_copy(
                acc.at[pl.ds(z0, ZR)],
                out_hbm.at[pl.ds((c * n_chunks + ch) * ACC_ROWS + z0, ZR)])
            plsc.subcore_barrier()

    out = agg_kernel(y, src_p, dst_p, zeros_fc)
    return out.reshape(NCORES, n_chunks, ACC_ROWS, FC)


def _tc_layer1_call(data, W, degp):
    fi = data.shape[1]

    def body(x_ref, w_ref, deg_ref, out_ref, dinv_ref):
        dinv = lax.rsqrt(deg_ref[0, :, 0:1] + deg_ref[1, :, 0:1] + 2.0)
        xw = jnp.dot(x_ref[...], w_ref[...], preferred_element_type=jnp.float32)
        out_ref[0] = dinv * xw
        dinv_ref[...] = jnp.broadcast_to(dinv, (RBLK, 8))

    return pl.pallas_call(
        body,
        grid=(N // RBLK,),
        in_specs=[
            pl.BlockSpec((RBLK, fi), lambda i: (i, 0)),
            pl.BlockSpec((fi, FC), lambda i: (0, 0)),
            pl.BlockSpec((2, RBLK, FC), lambda i: (0, i, 0)),
        ],
        out_specs=[
            pl.BlockSpec((1, RBLK, FC), lambda i: (0, i, 0)),
            pl.BlockSpec((RBLK, 8), lambda i: (i, 0)),
        ],
        out_shape=[
            jax.ShapeDtypeStruct((1, N, FC), jnp.float32),
            jax.ShapeDtypeStruct((N, 8), jnp.float32),
        ],
    )(data, W, degp)


def _tc_layer_call(aggp, y_prev, dinv8, b_prev, W):
    Cp = y_prev.shape[0]
    fi = Cp * FC
    fo = W.shape[1]
    Ck = fo // FC

    def body(a_ref, y_ref, deg_ref, b_ref, w_ref, out_ref):
        dinv = deg_ref[:, 0:1]
        parts = []
        for cc in range(Cp):
            a = a_ref[0, cc] + a_ref[1, cc]
            h = dinv * (a + 2.0 * y_ref[cc]) + b_ref[0, cc * FC:(cc + 1) * FC]
            parts.append(h)
        h = jnp.concatenate(parts, axis=1) if Cp > 1 else parts[0]
        h = jnp.maximum(h, 0.0)
        xw = jnp.dot(h, w_ref[...], preferred_element_type=jnp.float32)
        yk = dinv * xw
        for c2 in range(Ck):
            out_ref[c2] = yk[:, c2 * FC:(c2 + 1) * FC]

    return pl.pallas_call(
        body,
        grid=(N // RBLK,),
        in_specs=[
            pl.BlockSpec((2, Cp, RBLK, FC), lambda i: (0, 0, i, 0)),
            pl.BlockSpec((Cp, RBLK, FC), lambda i: (0, i, 0)),
            pl.BlockSpec((RBLK, 8), lambda i: (i, 0)),
            pl.BlockSpec((1, fi), lambda i: (0, 0)),
            pl.BlockSpec((fi, fo), lambda i: (0, 0)),
        ],
        out_specs=pl.BlockSpec((Ck, RBLK, FC), lambda i: (0, i, 0)),
        out_shape=jax.ShapeDtypeStruct((Ck, N, FC), jnp.float32),
    )(aggp, y_prev, dinv8, b_prev, W)


def _tc_final_call(aggp, y_prev, dinv8, b_prev, batch2d):
    def body(a_ref, y_ref, deg_ref, b_ref, bt_ref, out_ref):
        i = pl.program_id(0)

        @pl.when(i == 0)
        def _():
            out_ref[...] = jnp.full((NUM_GRAPHS, FC), -jnp.inf, jnp.float32)

        dinv = deg_ref[:, 0:1]
        a = a_ref[0, 0] + a_ref[1, 0]
        h = dinv * (a + 2.0 * y_ref[0]) + b_ref[0, :]
        bt = bt_ref[:, 0:1]
        bmin = jnp.min(bt)
        bmax = jnp.max(bt)
        for g in range(NUM_GRAPHS):
            @pl.when((g >= bmin) & (g <= bmax))
            def _():
                mv = jnp.max(jnp.where(bt == g, h, -jnp.inf),
                             axis=0, keepdims=True)
                out_ref[g:g + 1, :] = jnp.maximum(out_ref[g:g + 1, :], mv)

    return pl.pallas_call(
        body,
        grid=(N // RBLK,),
        in_specs=[
            pl.BlockSpec((2, 1, RBLK, FC), lambda i: (0, 0, i, 0)),
            pl.BlockSpec((1, RBLK, FC), lambda i: (0, i, 0)),
            pl.BlockSpec((RBLK, 8), lambda i: (i, 0)),
            pl.BlockSpec((1, FC), lambda i: (0, 0)),
            pl.BlockSpec((RBLK, 1), lambda i: (i, 0)),
        ],
        out_specs=pl.BlockSpec((NUM_GRAPHS, FC), lambda i: (0, 0)),
        out_shape=jax.ShapeDtypeStruct((NUM_GRAPHS, FC), jnp.float32),
    )(aggp, y_prev, dinv8, b_prev, batch2d)


def kernel(data, edge_index, batch, W1, b1, W2, b2, W3, b3, W4, b4, W5, b5):
    pad = E_PAD - E
    filler = jnp.stack([jnp.arange(pad, dtype=jnp.int32),
                        jnp.full((pad,), N, jnp.int32)])
    ep = jnp.concatenate([edge_index, filler], axis=1).reshape(2, NBLK, BLK)
    src_p = ep[0]
    dst_p = ep[1]
    ones128 = jnp.ones((BLK, FC), jnp.float32)
    zeros_fc = jnp.zeros((ZB, FC), jnp.float32)
    batch2d = batch.reshape(N, 1)
    b1r, b2r, b3r, b4r, b5r = (b.reshape(1, -1) for b in (b1, b2, b3, b4, b5))

    degp = _sc_degree_call(dst_p, ones128, zeros_fc)
    y1, dinv8 = _tc_layer1_call(data, W1, degp)
    agg1 = _sc_agg_call(1, y1, src_p, dst_p, zeros_fc)
    y2 = _tc_layer_call(agg1, y1, dinv8, b1r, W2)
    agg2 = _sc_agg_call(2, y2, src_p, dst_p, zeros_fc)
    y3 = _tc_layer_call(agg2, y2, dinv8, b2r, W3)
    agg3 = _sc_agg_call(4, y3, src_p, dst_p, zeros_fc)
    y4 = _tc_layer_call(agg3, y3, dinv8, b3r, W4)
    agg4 = _sc_agg_call(2, y4, src_p, dst_p, zeros_fc)
    y5 = _tc_layer_call(agg4, y4, dinv8, b4r, W5)
    agg5 = _sc_agg_call(1, y5, src_p, dst_p, zeros_fc)
    return _tc_final_call(agg5, y5, dinv8, b5r, batch2d)


# speedup vs baseline: 1.0030x; 1.0030x over previous
import functools

import jax
import jax.numpy as jnp
from jax import lax
from jax.experimental import pallas as pl
from jax.experimental.pallas import tpu as pltpu
from jax.experimental.pallas import tpu_sc as plsc

N = 10000
E = 320000
NUM_GRAPHS = 64
FC = 128
BLK = 128
NCORES = 2
NSUB = 16
NBLK = 2560
E_PAD = NBLK * BLK
BLK_PER_SUB = NBLK // (NCORES * NSUB)
ACC_ROWS = 10112
ZR = ACC_ROWS // NSUB
RBLK = 1000
IDXG = 16
ZB = 64


def _vector_mesh():
    return plsc.VectorSubcoreMesh(core_axis_name="c", subcore_axis_name="s")


def _zero_rows(zbuf, acc, z0):
    nfull = ZR // ZB
    for k in range(nfull):
        pltpu.sync_copy(zbuf, acc.at[pl.ds(z0 + k * ZB, ZB)])
    rem = ZR - nfull * ZB
    if rem:
        pltpu.sync_copy(zbuf.at[pl.ds(0, rem)],
                        acc.at[pl.ds(z0 + nfull * ZB, rem)])


def _sc_degree_call(dst_p, ones128, zeros_fc):
    @functools.partial(
        pl.kernel,
        out_type=jax.ShapeDtypeStruct((NCORES * ACC_ROWS, FC), jnp.float32),
        mesh=_vector_mesh(),
        scratch_types=[
            pltpu.VMEM((BLK_PER_SUB, BLK), jnp.int32),
            pltpu.VMEM((BLK, FC), jnp.float32),
            pltpu.VMEM((ZB, FC), jnp.float32),
            pltpu.VMEM_SHARED((ACC_ROWS, FC), jnp.float32),
            pltpu.SemaphoreType.DMA,
        ],
    )
    def deg_kernel(dst_hbm, ones_hbm, zeros_hbm, out_hbm, dst_v, ones_v,
                   zbuf, acc, ssem):
        c = lax.axis_index("c")
        s = lax.axis_index("s")
        b0 = (c * NSUB + s) * BLK_PER_SUB
        pltpu.sync_copy(dst_hbm.at[pl.ds(b0, BLK_PER_SUB)], dst_v)
        pltpu.sync_copy(ones_hbm, ones_v)
        pltpu.sync_copy(zeros_hbm, zbuf)
        z0 = s * ZR
        _zero_rows(zbuf, acc, z0)
        plsc.subcore_barrier()

        @pl.loop(0, BLK_PER_SUB, step=8)
        def _(j):
            for k in range(8):
                pltpu.async_copy(ones_v, acc.at[dst_v.at[j + k]], ssem,
                                 add=True)
            for k in range(8):
                pltpu.make_async_copy(ones_hbm, ones_v, ssem).wait()

        plsc.subcore_barrier()
        pltpu.sync_copy(acc.at[pl.ds(z0, ZR)],
                        out_hbm.at[pl.ds(c * ACC_ROWS + z0, ZR)])

    return deg_kernel(dst_p, ones128, zeros_fc).reshape(NCORES, ACC_ROWS, FC)


def _sc_agg_call(n_chunks, y, src_p, dst_p, zeros_fc):
    @functools.partial(
        pl.kernel,
        out_type=jax.ShapeDtypeStruct(
            (NCORES * n_chunks * ACC_ROWS, FC), jnp.float32),
        mesh=_vector_mesh(),
        scratch_types=[
            pltpu.VMEM((IDXG, BLK), jnp.int32),
            pltpu.VMEM((IDXG, BLK), jnp.int32),
            pltpu.VMEM((BLK, FC), jnp.float32),
            pltpu.VMEM((BLK, FC), jnp.float32),
            pltpu.VMEM((ZB, FC), jnp.float32),
            pltpu.VMEM_SHARED((ACC_ROWS, FC), jnp.float32),
            pltpu.SemaphoreType.DMA,
            pltpu.SemaphoreType.DMA,
            pltpu.SemaphoreType.DMA,
            pltpu.SemaphoreType.DMA,
            pltpu.SemaphoreType.DMA,
        ],
    )
    def agg_kernel(y_hbm, src_hbm, dst_hbm, zeros_hbm, out_hbm,
                   src_v, dst_v, rows0, rows1, zbuf, acc,
                   gsem0, gsem1, ssem0, ssem1, isem):
        c = lax.axis_index("c")
        s = lax.axis_index("s")
        n_grp = BLK_PER_SUB // IDXG
        base = (c * NSUB + s) * BLK_PER_SUB
        pltpu.sync_copy(zeros_hbm, zbuf)

        def drain_scatters():
            pltpu.make_async_copy(y_hbm.at[0].at[pl.ds(0, BLK)], rows0,
                                  ssem0).wait()
            pltpu.make_async_copy(y_hbm.at[0].at[pl.ds(0, BLK)], rows1,
                                  ssem1).wait()

        for ch in range(n_chunks):
            z0 = s * ZR
            _zero_rows(zbuf, acc, z0)
            plsc.subcore_barrier()

            @pl.loop(0, n_grp)
            def _(g):
                @pl.when(g > 0)
                def _():
                    drain_scatters()

                gb = base + g * IDXG
                ld0 = pltpu.async_copy(src_hbm.at[pl.ds(gb, IDXG)], src_v,
                                       isem)
                ld1 = pltpu.async_copy(dst_hbm.at[pl.ds(gb, IDXG)], dst_v,
                                       isem)
                ld0.wait()
                ld1.wait()
                for j in range(0, IDXG, 2):
                    if j > 0:
                        drain_scatters()
                    cp0 = pltpu.async_copy(
                        y_hbm.at[ch].at[src_v.at[j]], rows0, gsem0)
                    cp1 = pltpu.async_copy(
                        y_hbm.at[ch].at[src_v.at[j + 1]], rows1, gsem1)
                    cp0.wait()
                    pltpu.async_copy(rows0, acc.at[dst_v.at[j]], ssem0,
                                     add=True)
                    cp1.wait()
                    pltpu.async_copy(rows1, acc.at[dst_v.at[j + 1]], ssem1,
                                     add=True)

            drain_scatters()
            plsc.subcore_barrier()
            pltpu.sync_copy(
                acc.at[pl.ds(z0, ZR)],
                out_hbm.at[pl.ds((c * n_chunks + ch) * ACC_ROWS + z0, ZR)])
            plsc.subcore_barrier()

    out = agg_kernel(y, src_p, dst_p, zeros_fc)
    return out.reshape(NCORES, n_chunks, ACC_ROWS, FC)


def _tc_layer1_call(data, W, degp):
    fi = data.shape[1]

    def body(x_ref, w_ref, deg_ref, out_ref, dinv_ref):
        dinv = lax.rsqrt(deg_ref[0, :, 0:1] + deg_ref[1, :, 0:1] + 2.0)
        xw = jnp.dot(x_ref[...], w_ref[...], preferred_element_type=jnp.float32)
        out_ref[0] = dinv * xw
        dinv_ref[...] = jnp.broadcast_to(dinv, (RBLK, 8))

    return pl.pallas_call(
        body,
        grid=(N // RBLK,),
        in_specs=[
            pl.BlockSpec((RBLK, fi), lambda i: (i, 0)),
            pl.BlockSpec((fi, FC), lambda i: (0, 0)),
            pl.BlockSpec((2, RBLK, FC), lambda i: (0, i, 0)),
        ],
        out_specs=[
            pl.BlockSpec((1, RBLK, FC), lambda i: (0, i, 0)),
            pl.BlockSpec((RBLK, 8), lambda i: (i, 0)),
        ],
        out_shape=[
            jax.ShapeDtypeStruct((1, N, FC), jnp.float32),
            jax.ShapeDtypeStruct((N, 8), jnp.float32),
        ],
    )(data, W, degp)


def _tc_layer_call(aggp, y_prev, dinv8, b_prev, W):
    Cp = y_prev.shape[0]
    fi = Cp * FC
    fo = W.shape[1]
    Ck = fo // FC

    def body(a_ref, y_ref, deg_ref, b_ref, w_ref, out_ref):
        dinv = deg_ref[:, 0:1]
        parts = []
        for cc in range(Cp):
            a = a_ref[0, cc] + a_ref[1, cc]
            h = dinv * (a + 2.0 * y_ref[cc]) + b_ref[0, cc * FC:(cc + 1) * FC]
            parts.append(h)
        h = jnp.concatenate(parts, axis=1) if Cp > 1 else parts[0]
        h = jnp.maximum(h, 0.0)
        xw = jnp.dot(h, w_ref[...], preferred_element_type=jnp.float32)
        yk = dinv * xw
        for c2 in range(Ck):
            out_ref[c2] = yk[:, c2 * FC:(c2 + 1) * FC]

    return pl.pallas_call(
        body,
        grid=(N // RBLK,),
        in_specs=[
            pl.BlockSpec((2, Cp, RBLK, FC), lambda i: (0, 0, i, 0)),
            pl.BlockSpec((Cp, RBLK, FC), lambda i: (0, i, 0)),
            pl.BlockSpec((RBLK, 8), lambda i: (i, 0)),
            pl.BlockSpec((1, fi), lambda i: (0, 0)),
            pl.BlockSpec((fi, fo), lambda i: (0, 0)),
        ],
        out_specs=pl.BlockSpec((Ck, RBLK, FC), lambda i: (0, i, 0)),
        out_shape=jax.ShapeDtypeStruct((Ck, N, FC), jnp.float32),
    )(aggp, y_prev, dinv8, b_prev, W)


def _tc_final_call(aggp, y_prev, dinv8, b_prev, batch2d):
    def body(a_ref, y_ref, deg_ref, b_ref, bt_ref, out_ref):
        i = pl.program_id(0)

        @pl.when(i == 0)
        def _():
            out_ref[...] = jnp.full((NUM_GRAPHS, FC), -jnp.inf, jnp.float32)

        dinv = deg_ref[:, 0:1]
        a = a_ref[0, 0] + a_ref[1, 0]
        h = dinv * (a + 2.0 * y_ref[0]) + b_ref[0, :]
        bt = bt_ref[:, 0:1]
        bmin = jnp.min(bt)
        bmax = jnp.max(bt)
        for g in range(NUM_GRAPHS):
            @pl.when((g >= bmin) & (g <= bmax))
            def _():
                mv = jnp.max(jnp.where(bt == g, h, -jnp.inf),
                             axis=0, keepdims=True)
                out_ref[g:g + 1, :] = jnp.maximum(out_ref[g:g + 1, :], mv)

    return pl.pallas_call(
        body,
        grid=(N // RBLK,),
        in_specs=[
            pl.BlockSpec((2, 1, RBLK, FC), lambda i: (0, 0, i, 0)),
            pl.BlockSpec((1, RBLK, FC), lambda i: (0, i, 0)),
            pl.BlockSpec((RBLK, 8), lambda i: (i, 0)),
            pl.BlockSpec((1, FC), lambda i: (0, 0)),
            pl.BlockSpec((RBLK, 1), lambda i: (i, 0)),
        ],
        out_specs=pl.BlockSpec((NUM_GRAPHS, FC), lambda i: (0, 0)),
        out_shape=jax.ShapeDtypeStruct((NUM_GRAPHS, FC), jnp.float32),
    )(aggp, y_prev, dinv8, b_prev, batch2d)


def kernel(data, edge_index, batch, W1, b1, W2, b2, W3, b3, W4, b4, W5, b5):
    pad = E_PAD - E
    filler = jnp.stack([jnp.arange(pad, dtype=jnp.int32),
                        jnp.full((pad,), N, jnp.int32)])
    ep = jnp.concatenate([edge_index, filler], axis=1).reshape(2, NBLK, BLK)
    src_p = ep[0]
    dst_p = ep[1]
    ones128 = jnp.ones((BLK, FC), jnp.float32)
    zeros_fc = jnp.zeros((ZB, FC), jnp.float32)
    batch2d = batch.reshape(N, 1)
    b1r, b2r, b3r, b4r, b5r = (b.reshape(1, -1) for b in (b1, b2, b3, b4, b5))

    degp = _sc_degree_call(dst_p, ones128, zeros_fc)
    y1, dinv8 = _tc_layer1_call(data, W1, degp)
    agg1 = _sc_agg_call(1, y1, src_p, dst_p, zeros_fc)
    y2 = _tc_layer_call(agg1, y1, dinv8, b1r, W2)
    agg2 = _sc_agg_call(2, y2, src_p, dst_p, zeros_fc)
    y3 = _tc_layer_call(agg2, y2, dinv8, b2r, W3)
    agg3 = _sc_agg_call(4, y3, src_p, dst_p, zeros_fc)
    y4 = _tc_layer_call(agg3, y3, dinv8, b3r, W4)
    agg4 = _sc_agg_call(2, y4, src_p, dst_p, zeros_fc)
    y5 = _tc_layer_call(agg4, y4, dinv8, b4r, W5)
    agg5 = _sc_agg_call(1, y5, src_p, dst_p, zeros_fc)
    return _tc_final_call(agg5, y5, dinv8, b5r, batch2d)
